# baseline (speedup 1.0000x reference)
import functools

import jax
import jax.numpy as jnp
from jax import lax
from jax.experimental import pallas as pl
from jax.experimental.pallas import tpu as pltpu
from jax.experimental.pallas import tpu_sc as plsc

_CUTOFF = 5.0
_W1 = 512
_W = 1024


def _mesh():
    return plsc.VectorSubcoreMesh(core_axis_name="c", subcore_axis_name="s")


_CP = pltpu.CompilerParams(use_tc_tiling_on_sc=False,
                           needs_layout_passes=False)


def _build_node_tab(table16, anum2d):
    n_nodes = anum2d.shape[1]

    @functools.partial(
        pl.kernel,
        out_type=jax.ShapeDtypeStruct((n_nodes, 16), jnp.float32),
        mesh=_mesh(),
        compiler_params=_CP,
    )
    def stage1(table_hbm, anum_hbm, out_hbm):
        def body(a_v, o_v):
            pltpu.sync_copy(table_hbm.at[a_v.at[0]], o_v)

        pltpu.emit_pipeline(
            body,
            grid=(n_nodes // _W1,),
            in_specs=[pl.BlockSpec((1, _W1), lambda i: (0, i))],
            out_specs=[pl.BlockSpec((_W1, 16), lambda i: (i, 0))],
            core_axis_name=("c", "s"),
            dimension_semantics=(pltpu.PARALLEL,),
        )(anum_hbm, out_hbm)

    return stage1(table16, anum2d)


def _radial(node_tab, idx_i, idx_j, dist):
    n_edges = idx_i.shape[1]

    @functools.partial(
        pl.kernel,
        out_type=jax.ShapeDtypeStruct((n_edges * 27,), jnp.float32),
        mesh=_mesh(),
        compiler_params=_CP,
        scratch_types=[
            pltpu.VMEM((_W, 16), jnp.float32),
            pltpu.VMEM((_W, 16), jnp.float32),
            pltpu.VMEM((_W,), jnp.float32),
            pltpu.SemaphoreType.DMA,
            pltpu.SemaphoreType.DMA,
        ],
    )
    def stage2(tab_hbm, ii_hbm, jj_hbm, dd_hbm, out_hbm, gi_v, gj_v, rec_v,
               sem1, sem2):
        iota = lax.broadcasted_iota(jnp.int32, (16,), 0)
        lt12 = iota < 12

        def body(ii_v, jj_v, dd_v, out_v):
            cp1 = pltpu.async_copy(tab_hbm.at[ii_v.at[0]], gi_v, sem1)
            cp2 = pltpu.async_copy(tab_hbm.at[jj_v.at[0]], gj_v, sem2)
            cp1.wait()
            cp2.wait()

            @pl.loop(0, _W, step=16)
            def _(k0):
                sl16 = pl.ds(k0, 16)
                dvec = dd_v.at[0][sl16]
                rec = 1.0 / dvec
                rec_v[sl16] = rec
                base = (k0 + iota) * 27
                plsc.store_scatter(out_v, [base], dvec)
                plsc.store_scatter(out_v, [base + 25],
                                   dvec * dvec * (1.0 / _CUTOFF))
                plsc.store_scatter(out_v, [base + 26], _CUTOFF * rec)

            @pl.loop(0, _W)
            def _(r):
                sr = jnp.full((16,), r, jnp.int32)
                gi_row = plsc.load_gather(gi_v, [sr, iota])
                gj_row = plsc.load_gather(gj_v, [sr, iota])
                recb = plsc.load_gather(rec_v, [sr])
                pos = r * 27 + 1 + iota
                plsc.store_scatter(out_v, [pos], gi_row, mask=lt12)
                plsc.store_scatter(out_v, [pos + 12], gj_row * recb,
                                   mask=lt12)

        pltpu.emit_pipeline(
            body,
            grid=(n_edges // _W,),
            in_specs=[
                pl.BlockSpec((1, _W), lambda i: (0, i)),
                pl.BlockSpec((1, _W), lambda i: (0, i)),
                pl.BlockSpec((1, _W), lambda i: (0, i)),
            ],
            out_specs=[pl.BlockSpec((_W * 27,), lambda i: (i,))],
            core_axis_name=("c", "s"),
            dimension_semantics=(pltpu.PARALLEL,),
        )(ii_hbm, jj_hbm, dd_hbm, out_hbm)

    return stage2(node_tab, idx_i, idx_j, dist)


def kernel(atoms_long, atom_embed_table, batch_atom_ij_idx, batch_dist_ij):
    n_edges = batch_atom_ij_idx.shape[0]
    n_nodes = atoms_long.shape[0]
    n_pad = -(-n_nodes // _W1) * _W1
    anum2d = jnp.pad(atoms_long[:, 1].astype(jnp.int32),
                     (0, n_pad - n_nodes)).reshape(1, -1)
    table16 = jnp.pad(atom_embed_table, ((0, 0), (0, 4)))
    ij32 = batch_atom_ij_idx.astype(jnp.int32)
    idx_i = ij32[:, 0].reshape(1, n_edges)
    idx_j = ij32[:, 1].reshape(1, n_edges)
    dist2d = batch_dist_ij.reshape(1, n_edges)

    node_tab = _build_node_tab(table16, anum2d)
    n_chunks = 5
    ck = n_edges // n_chunks
    pieces = []
    for k in range(n_chunks):
        sl = slice(k * ck, (k + 1) * ck)
        pieces.append(_radial(node_tab, idx_i[:, sl], idx_j[:, sl],
                              dist2d[:, sl]).reshape(ck, 27))
    rad_desc = jnp.concatenate(pieces, axis=0)
    return rad_desc, batch_atom_ij_idx[:, 0]


# speedup vs baseline: 1.2297x; 1.2297x over previous
import functools

import jax
import jax.numpy as jnp
from jax import lax
from jax.experimental import pallas as pl
from jax.experimental.pallas import tpu as pltpu
from jax.experimental.pallas import tpu_sc as plsc

_CUTOFF = 5.0
_W1 = 512
_W = 1024


def _mesh():
    return plsc.VectorSubcoreMesh(core_axis_name="c", subcore_axis_name="s")


_CP = pltpu.CompilerParams(use_tc_tiling_on_sc=False,
                           needs_layout_passes=False)


def _build_node_tab(table16, anum2d):
    n_nodes = anum2d.shape[1]

    @functools.partial(
        pl.kernel,
        out_type=jax.ShapeDtypeStruct((n_nodes, 16), jnp.float32),
        mesh=_mesh(),
        compiler_params=_CP,
    )
    def stage1(table_hbm, anum_hbm, out_hbm):
        def body(a_v, o_v):
            pltpu.sync_copy(table_hbm.at[a_v.at[0]], o_v)

        pltpu.emit_pipeline(
            body,
            grid=(n_nodes // _W1,),
            in_specs=[pl.BlockSpec((1, _W1), lambda i: (0, i))],
            out_specs=[pl.BlockSpec((_W1, 16), lambda i: (i, 0))],
            core_axis_name=("c", "s"),
            dimension_semantics=(pltpu.PARALLEL,),
        )(anum_hbm, out_hbm)

    return stage1(table16, anum2d)


def _radial(node_tab, idx_i, idx_j, dist):
    n_edges = idx_i.shape[1]

    @functools.partial(
        pl.kernel,
        out_type=jax.ShapeDtypeStruct((n_edges, 27), jnp.float32),
        mesh=_mesh(),
        compiler_params=_CP,
        scratch_types=[
            pltpu.VMEM((_W, 16), jnp.float32),
            pltpu.VMEM((_W, 16), jnp.float32),
            pltpu.VMEM((_W,), jnp.float32),
            pltpu.SemaphoreType.DMA,
            pltpu.SemaphoreType.DMA,
        ],
    )
    def stage2(tab_hbm, ii_hbm, jj_hbm, dd_hbm, out_hbm, gi_v, gj_v, rec_v,
               sem1, sem2):
        iota = lax.broadcasted_iota(jnp.int32, (16,), 0)
        lt12 = iota < 12

        def body(ii_v, jj_v, dd_v, out_v):
            cp1 = pltpu.async_copy(tab_hbm.at[ii_v.at[0]], gi_v, sem1)
            cp2 = pltpu.async_copy(tab_hbm.at[jj_v.at[0]], gj_v, sem2)
            cp1.wait()
            cp2.wait()

            @pl.loop(0, _W, step=16)
            def _(k0):
                sl16 = pl.ds(k0, 16)
                dvec = dd_v.at[0][sl16]
                rec = 1.0 / dvec
                rec_v[sl16] = rec
                rows = k0 + iota
                zero = iota * 0
                plsc.store_scatter(out_v, [rows, zero], dvec)
                plsc.store_scatter(out_v, [rows, zero + 25],
                                   dvec * dvec * (1.0 / _CUTOFF))
                plsc.store_scatter(out_v, [rows, zero + 26], _CUTOFF * rec)

            @pl.loop(0, _W)
            def _(r):
                sr = jnp.full((16,), r, jnp.int32)
                gi_row = plsc.load_gather(gi_v, [sr, iota])
                gj_row = plsc.load_gather(gj_v, [sr, iota])
                recb = plsc.load_gather(rec_v, [sr])
                cols = iota + 1
                plsc.store_scatter(out_v, [sr, cols], gi_row, mask=lt12)
                plsc.store_scatter(out_v, [sr, cols + 12], gj_row * recb,
                                   mask=lt12)

        pltpu.emit_pipeline(
            body,
            grid=(n_edges // _W,),
            in_specs=[
                pl.BlockSpec((1, _W), lambda i: (0, i)),
                pl.BlockSpec((1, _W), lambda i: (0, i)),
                pl.BlockSpec((1, _W), lambda i: (0, i)),
            ],
            out_specs=[pl.BlockSpec((_W, 27), lambda i: (i, 0))],
            core_axis_name=("c", "s"),
            dimension_semantics=(pltpu.PARALLEL,),
        )(ii_hbm, jj_hbm, dd_hbm, out_hbm)

    return stage2(node_tab, idx_i, idx_j, dist)


def kernel(atoms_long, atom_embed_table, batch_atom_ij_idx, batch_dist_ij):
    n_edges = batch_atom_ij_idx.shape[0]
    n_nodes = atoms_long.shape[0]
    n_pad = -(-n_nodes // _W1) * _W1
    anum2d = jnp.pad(atoms_long[:, 1].astype(jnp.int32),
                     (0, n_pad - n_nodes)).reshape(1, -1)
    table16 = jnp.pad(atom_embed_table, ((0, 0), (0, 4)))
    ij32 = batch_atom_ij_idx.astype(jnp.int32)
    idx_i = ij32[:, 0].reshape(1, n_edges)
    idx_j = ij32[:, 1].reshape(1, n_edges)
    dist2d = batch_dist_ij.reshape(1, n_edges)

    node_tab = _build_node_tab(table16, anum2d)
    n_chunks = 5
    ck = n_edges // n_chunks
    pieces = []
    for k in range(n_chunks):
        sl = slice(k * ck, (k + 1) * ck)
        pieces.append(_radial(node_tab, idx_i[:, sl], idx_j[:, sl],
                              dist2d[:, sl]))
    rad_desc = jnp.concatenate(pieces, axis=0)
    return rad_desc, batch_atom_ij_idx[:, 0]
